# baseline (speedup 1.0000x reference)
import functools

import jax
import jax.numpy as jnp
from jax.experimental import pallas as pl

_ROWS = 256


def _logsoftmax_kernel(x_ref, wt_ref, b_ref, o_ref):
    logits = (
        jnp.dot(x_ref[...], wt_ref[...], preferred_element_type=jnp.float32)
        + b_ref[...]
    )
    m = jnp.max(logits, axis=1, keepdims=True)
    lse = jnp.log(jnp.sum(jnp.exp(logits - m), axis=1, keepdims=True))
    o_ref[...] = logits - m - lse


@jax.jit
def kernel(x, xl, W, b):
    B, T, D = x.shape
    V = W.shape[0]
    rows = B * T
    x2 = x.reshape(rows, D).astype(jnp.bfloat16)
    wt = W.T.astype(jnp.bfloat16)
    b2 = b.reshape(1, V)

    out = pl.pallas_call(
        _logsoftmax_kernel,
        grid=(rows // _ROWS,),
        in_specs=[
            pl.BlockSpec((_ROWS, D), lambda i: (i, 0)),
            pl.BlockSpec((D, V), lambda i: (0, 0)),
            pl.BlockSpec((1, V), lambda i: (0, 0)),
        ],
        out_specs=pl.BlockSpec((_ROWS, V), lambda i: (i, 0)),
        out_shape=jax.ShapeDtypeStruct((rows, V), jnp.float32),
    )(x2, wt, b2)
    return out.reshape(B, T, V)


# speedup vs baseline: 2.1186x; 2.1186x over previous
import jax
import jax.numpy as jnp
from jax.experimental import pallas as pl

_R = 256


def _logsoftmax_kernel(x_ref, w_ref, b_ref, o_ref):
    logits = (
        jnp.dot(w_ref[...], x_ref[0], preferred_element_type=jnp.float32)
        + b_ref[...]
    )
    m = jnp.max(logits, axis=0, keepdims=True)
    lse = jnp.log(jnp.sum(jnp.exp(logits - m), axis=0, keepdims=True))
    o_ref[0] = logits - m - lse


@jax.jit
def kernel(x, xl, W, b):
    B, T, D = x.shape
    V = W.shape[0]
    xt = x.transpose(0, 2, 1).astype(jnp.bfloat16)
    wb = W.astype(jnp.bfloat16)
    b2 = b.reshape(V, 1)

    out_t = pl.pallas_call(
        _logsoftmax_kernel,
        grid=(B, T // _R),
        in_specs=[
            pl.BlockSpec((1, D, _R), lambda bi, ti: (bi, 0, ti)),
            pl.BlockSpec((V, D), lambda bi, ti: (0, 0)),
            pl.BlockSpec((V, 1), lambda bi, ti: (0, 0)),
        ],
        out_specs=pl.BlockSpec((1, V, _R), lambda bi, ti: (bi, 0, ti)),
        out_shape=jax.ShapeDtypeStruct((B, V, T), jnp.float32),
    )(xt, wb, b2)
    return out_t.transpose(0, 2, 1)
